# baseline (speedup 1.0000x reference)
import jax
import jax.numpy as jnp
from jax import lax
from jax.experimental import pallas as pl
from jax.experimental.pallas import tpu as pltpu
from jax.experimental.pallas import tpu_sc as plsc

_MARGIN = 0.2
_K = 64
_L = 16
_NSUB = 16
_NCORE = 2
_NW = _NCORE * _NSUB


def _sc_topk_body(pos_ref, neg_ref, out_ref, row_v, kb_v, hist_v, vhist_v,
                  tot_v, vt_v, gtot_v, gv_v, res_v, sem):
    rows, n = pos_ref.shape
    nv = n // _L
    rpw = rows // _NW
    c = lax.axis_index("c")
    s = lax.axis_index("s")
    wid = c * _NSUB + s
    base = wid * rpw
    lanes = jnp.arange(_L, dtype=jnp.int32)
    ones = jnp.ones((_L,), jnp.int32)
    zeros_i = jnp.zeros((_L,), jnp.int32)
    zeros_f = jnp.zeros((_L,), jnp.float32)

    @plsc.parallel_loop(0, 256, unroll=8)
    def _(i):
        hist_v[pl.ds(i * _L, _L)] = zeros_i
        vhist_v[pl.ds(i * _L, _L)] = zeros_f

    def run(src_ref, negate, lane_off, res_vec0):
        pltpu.async_copy(src_ref.at[base], row_v.at[pl.ds(0, n)], sem.at[0])

        def task(j, res_vec):
            pj = lax.rem(j, 2)
            npj = 1 - pj

            @pl.when(j < rpw - 1)
            def _():
                pltpu.async_copy(src_ref.at[base + j + 1],
                                 row_v.at[pl.ds(npj * n, n)], sem.at[npj])

            pltpu.make_async_copy(src_ref.at[base + j],
                                  row_v.at[pl.ds(pj * n, n)], sem.at[pj]).wait()
            off = pj * n

            def load_v(i):
                v = row_v[pl.ds(off + i * _L, _L)]
                return -v if negate else v

            @plsc.parallel_loop(0, nv, unroll=8)
            def _(i):
                v = load_v(i)
                u = lax.bitcast_convert_type(v, jnp.uint32)
                m = jnp.uint32(0x80000000) | (jnp.uint32(0) - (u >> jnp.uint32(31)))
                kb = u ^ m
                kb_v[pl.ds(i * _L, _L)] = kb
                idx = (kb >> jnp.uint32(24)).astype(jnp.int32) * _L + lanes
                plsc.addupdate_scatter(hist_v, [idx], ones)
                plsc.addupdate_scatter(vhist_v, [idx], v)

            carry = (jnp.int32(_K), jnp.uint32(0), jnp.bool_(False),
                     jnp.float32(0.0))
            for p in range(4):
                shift = 24 - 8 * p
                r_in, prefix_in, done_in, vacc_in = carry

                if p > 0:
                    @pl.when(jnp.logical_not(done_in))
                    def _(_shift=shift, _prefix=prefix_in):
                        @plsc.parallel_loop(0, nv, unroll=8)
                        def _(i):
                            kb = kb_v[pl.ds(i * _L, _L)]
                            bucket = ((kb >> jnp.uint32(_shift))
                                      & jnp.uint32(0xFF)).astype(jnp.int32)
                            active = (kb >> jnp.uint32(_shift + 8)) == _prefix
                            idx = bucket * _L + lanes
                            plsc.addupdate_scatter(hist_v, [idx], ones,
                                                   mask=active)
                            plsc.addupdate_scatter(vhist_v, [idx], load_v(i),
                                                   mask=active)

                def red(g):
                    hs = [hist_v[pl.ds((g * _L + k) * _L, _L)] for k in range(_L)]
                    t01 = (hs[0] + hs[1]) + (hs[2] + hs[3])
                    t23 = (hs[4] + hs[5]) + (hs[6] + hs[7])
                    t45 = (hs[8] + hs[9]) + (hs[10] + hs[11])
                    t67 = (hs[12] + hs[13]) + (hs[14] + hs[15])
                    gtot_v[g] = jnp.sum((t01 + t23) + (t45 + t67))
                    vs = [vhist_v[pl.ds((g * _L + k) * _L, _L)] for k in range(_L)]
                    v01 = (vs[0] + vs[1]) + (vs[2] + vs[3])
                    v23 = (vs[4] + vs[5]) + (vs[6] + vs[7])
                    v45 = (vs[8] + vs[9]) + (vs[10] + vs[11])
                    v67 = (vs[12] + vs[13]) + (vs[14] + vs[15])
                    gv_v[g] = jnp.sum((v01 + v23) + (v45 + v67))

                if p == 0:
                    plsc.parallel_loop(0, 16)(red)
                else:
                    @pl.when(jnp.logical_not(done_in))
                    def _():
                        plsc.parallel_loop(0, 16)(red)

                def scang(i, cg, _r=r_in):
                    S, Sv, gsel, Ssel, Svsel, found = cg
                    g = 15 - i
                    Sn = S + gtot_v[g]
                    fn = jnp.logical_and(jnp.logical_not(found), Sn >= _r)
                    gsel = jnp.where(fn, g, gsel)
                    Ssel = jnp.where(fn, S, Ssel)
                    Svsel = jnp.where(fn, Sv, Svsel)
                    return (Sn, Sv + gv_v[g], gsel, Ssel, Svsel,
                            jnp.logical_or(found, fn))

                _, _, gsel, s_above, sv_above, _ = lax.fori_loop(
                    0, 16, scang,
                    (jnp.int32(0), jnp.float32(0.0), jnp.int32(0),
                     jnp.int32(0), jnp.float32(0.0), jnp.bool_(False)),
                    unroll=4)

                @plsc.parallel_loop(0, 16)
                def _(k):
                    tot_v[k] = jnp.sum(hist_v[pl.ds((gsel * _L + k) * _L, _L)])
                    vt_v[k] = jnp.sum(vhist_v[pl.ds((gsel * _L + k) * _L, _L)])

                @plsc.parallel_loop(0, 256, unroll=8)
                def _(i):
                    hist_v[pl.ds(i * _L, _L)] = zeros_i
                    vhist_v[pl.ds(i * _L, _L)] = zeros_f

                S = s_above
                vS = sv_above
                bsel = jnp.int32(0)
                sub = jnp.int32(0)
                totb = jnp.int32(0)
                vsub = jnp.float32(0.0)
                vtb = jnp.float32(0.0)
                found = jnp.bool_(False)
                for i in range(_L):
                    li = _L - 1 - i
                    cnt = tot_v[li]
                    val = vt_v[li]
                    Sn = S + cnt
                    fn = jnp.logical_and(jnp.logical_not(found), Sn >= r_in)
                    bsel = jnp.where(fn, gsel * _L + li, bsel)
                    sub = jnp.where(fn, S, sub)
                    vsub = jnp.where(fn, vS, vsub)
                    totb = jnp.where(fn, cnt, totb)
                    vtb = jnp.where(fn, val, vtb)
                    found = jnp.logical_or(found, fn)
                    S = Sn
                    vS = vS + val

                bsel = jnp.where(done_in, 0, bsel)
                sub = jnp.where(done_in, 0, sub)
                r_out = r_in - sub
                newly = jnp.logical_and(jnp.logical_not(done_in),
                                        totb == r_out)
                vacc_out = (vacc_in
                            + jnp.where(done_in, jnp.float32(0.0), vsub)
                            + jnp.where(newly, vtb, jnp.float32(0.0)))
                carry = (r_out,
                         (prefix_in << jnp.uint32(8)) | bsel.astype(jnp.uint32),
                         jnp.logical_or(done_in, newly),
                         vacc_out)

            r_f, kb_t, done_f, vacc = carry
            kb_vec = jnp.full((_L,), kb_t, dtype=jnp.uint32)
            was_pos = (kb_vec >> jnp.uint32(31)) == jnp.uint32(1)
            bits = jnp.where(was_pos, kb_vec ^ jnp.uint32(0x80000000), ~kb_vec)
            t_s = jnp.max(lax.bitcast_convert_type(bits, jnp.float32))
            tail = jnp.where(done_f, jnp.float32(0.0),
                             r_f.astype(jnp.float32) * t_s)
            res = (vacc + tail) * jnp.float32(1.0 / _K)
            if negate:
                res = -res
            return jnp.where(lanes == lane_off + j, res, res_vec)

        return lax.fori_loop(0, rpw, task, res_vec0)

    res_vec = run(pos_ref, False, 0, jnp.zeros((_L,), jnp.float32))
    res_vec = run(neg_ref, True, rpw, res_vec)
    res_v[...] = res_vec
    pltpu.sync_copy(res_v, out_ref.at[wid])


def _combine(packed, rows):
    rpw = rows // _NW

    def body(x_ref, o_ref):
        x = x_ref[...]
        hp = x[:, 0:rpw]
        hn = x[:, rpw:2 * rpw]
        loss = jnp.maximum(hp - hn + jnp.float32(_MARGIN), 0.0)
        o_ref[...] = jnp.reshape(jnp.sum(loss) * jnp.float32(1.0 / rows), (1, 1))

    return pl.pallas_call(
        body, out_shape=jax.ShapeDtypeStruct((1, 1), jnp.float32))(packed)[0, 0]


@jax.jit
def kernel(positive_distances, negative_distances):
    rows, n = positive_distances.shape
    mesh = plsc.VectorSubcoreMesh(core_axis_name="c", subcore_axis_name="s")
    sc_fn = pl.kernel(
        _sc_topk_body,
        mesh=mesh,
        compiler_params=pltpu.CompilerParams(needs_layout_passes=False),
        out_type=jax.ShapeDtypeStruct((_NW, _L), jnp.float32),
        scratch_types=[
            pltpu.VMEM((2 * n,), jnp.float32),
            pltpu.VMEM((n,), jnp.uint32),
            pltpu.VMEM((_L * 256,), jnp.int32),
            pltpu.VMEM((_L * 256,), jnp.float32),
            pltpu.SMEM((16,), jnp.int32),
            pltpu.SMEM((16,), jnp.float32),
            pltpu.SMEM((16,), jnp.int32),
            pltpu.SMEM((16,), jnp.float32),
            pltpu.VMEM((_L,), jnp.float32),
            pltpu.SemaphoreType.DMA((2,)),
        ],
    )
    packed = sc_fn(positive_distances, negative_distances)
    return _combine(packed, rows)


# speedup vs baseline: 1.4383x; 1.4383x over previous
import jax
import jax.numpy as jnp
from jax import lax
from jax.experimental import pallas as pl
from jax.experimental.pallas import tpu as pltpu
from jax.experimental.pallas import tpu_sc as plsc

_MARGIN = 0.2
_K = 64
_L = 16
_NSUB = 16
_NCORE = 2
_NW = _NCORE * _NSUB


def _sc_topk_body(pos_ref, neg_ref, out_ref, row_v, kb_v, hist_v, tot_v,
                  gtot_v, res_v, sem):
    rows, n = pos_ref.shape
    nv = n // _L
    rpw = rows // _NW
    c = lax.axis_index("c")
    s = lax.axis_index("s")
    wid = c * _NSUB + s
    base = wid * rpw
    lanes = jnp.arange(_L, dtype=jnp.int32)
    ones = jnp.ones((_L,), jnp.int32)
    zeros_i = jnp.zeros((_L,), jnp.int32)

    @plsc.parallel_loop(0, 256, unroll=8)
    def _(i):
        hist_v[pl.ds(i * _L, _L)] = zeros_i

    def run(src_ref, negate, lane_off, res_vec0):
        pltpu.async_copy(src_ref.at[base], row_v.at[pl.ds(0, n)], sem.at[0])

        def task(j, res_vec):
            pj = lax.rem(j, 2)
            npj = 1 - pj

            @pl.when(j < rpw - 1)
            def _():
                pltpu.async_copy(src_ref.at[base + j + 1],
                                 row_v.at[pl.ds(npj * n, n)], sem.at[npj])

            pltpu.make_async_copy(src_ref.at[base + j],
                                  row_v.at[pl.ds(pj * n, n)], sem.at[pj]).wait()
            off = pj * n

            def load_v(i):
                v = row_v[pl.ds(off + i * _L, _L)]
                return -v if negate else v

            @plsc.parallel_loop(0, nv, unroll=8)
            def _(i):
                u = lax.bitcast_convert_type(load_v(i), jnp.uint32)
                m = jnp.uint32(0x80000000) | (jnp.uint32(0) - (u >> jnp.uint32(31)))
                kb = u ^ m
                kb_v[pl.ds(i * _L, _L)] = kb
                bucket = (kb >> jnp.uint32(24)).astype(jnp.int32)
                plsc.addupdate_scatter(hist_v, [bucket * _L + lanes], ones)

            carry = (jnp.int32(_K), jnp.uint32(0), jnp.bool_(False))
            for p in range(4):
                shift = 24 - 8 * p
                r_in, prefix_in, done_in = carry

                if p > 0:
                    @pl.when(jnp.logical_not(done_in))
                    def _(_shift=shift, _prefix=prefix_in):
                        @plsc.parallel_loop(0, nv, unroll=8)
                        def _(i):
                            kb = kb_v[pl.ds(i * _L, _L)]
                            bucket = ((kb >> jnp.uint32(_shift))
                                      & jnp.uint32(0xFF)).astype(jnp.int32)
                            active = (kb >> jnp.uint32(_shift + 8)) == _prefix
                            plsc.addupdate_scatter(hist_v, [bucket * _L + lanes],
                                                   ones, mask=active)

                def red(g):
                    hs = [hist_v[pl.ds((g * _L + k) * _L, _L)] for k in range(_L)]
                    t01 = (hs[0] + hs[1]) + (hs[2] + hs[3])
                    t23 = (hs[4] + hs[5]) + (hs[6] + hs[7])
                    t45 = (hs[8] + hs[9]) + (hs[10] + hs[11])
                    t67 = (hs[12] + hs[13]) + (hs[14] + hs[15])
                    gtot_v[g] = jnp.sum((t01 + t23) + (t45 + t67))

                if p == 0:
                    plsc.parallel_loop(0, 16)(red)
                else:
                    @pl.when(jnp.logical_not(done_in))
                    def _():
                        plsc.parallel_loop(0, 16)(red)

                def scang(i, cg, _r=r_in):
                    S, gsel, Ssel, found = cg
                    g = 15 - i
                    Sn = S + gtot_v[g]
                    fn = jnp.logical_and(jnp.logical_not(found), Sn >= _r)
                    gsel = jnp.where(fn, g, gsel)
                    Ssel = jnp.where(fn, S, Ssel)
                    return (Sn, gsel, Ssel, jnp.logical_or(found, fn))

                _, gsel, s_above, _ = lax.fori_loop(
                    0, 16, scang,
                    (jnp.int32(0), jnp.int32(0), jnp.int32(0), jnp.bool_(False)),
                    unroll=4)

                @plsc.parallel_loop(0, 16)
                def _(k):
                    tot_v[k] = jnp.sum(hist_v[pl.ds((gsel * _L + k) * _L, _L)])

                @plsc.parallel_loop(0, 256, unroll=8)
                def _(i):
                    hist_v[pl.ds(i * _L, _L)] = zeros_i

                S = s_above
                bsel = jnp.int32(0)
                sub = jnp.int32(0)
                totb = jnp.int32(0)
                found = jnp.bool_(False)
                for i in range(_L):
                    li = _L - 1 - i
                    cnt = tot_v[li]
                    Sn = S + cnt
                    fn = jnp.logical_and(jnp.logical_not(found), Sn >= r_in)
                    bsel = jnp.where(fn, gsel * _L + li, bsel)
                    sub = jnp.where(fn, S, sub)
                    totb = jnp.where(fn, cnt, totb)
                    found = jnp.logical_or(found, fn)
                    S = Sn

                bsel = jnp.where(done_in, 0, bsel)
                sub = jnp.where(done_in, 0, sub)
                r_out = r_in - sub
                carry = (r_out,
                         (prefix_in << jnp.uint32(8)) | bsel.astype(jnp.uint32),
                         jnp.logical_or(done_in, totb == r_out))

            _, kb_t, _ = carry
            kb_vec = jnp.full((_L,), kb_t, dtype=jnp.uint32)
            was_pos = (kb_vec >> jnp.uint32(31)) == jnp.uint32(1)
            bits = jnp.where(was_pos, kb_vec ^ jnp.uint32(0x80000000), ~kb_vec)
            t_vec = lax.bitcast_convert_type(bits, jnp.float32)

            def sb(i, acc):
                parts = [jnp.maximum(load_v(i + k) - t_vec, jnp.float32(0.0))
                         for k in range(8)]
                s01 = (parts[0] + parts[1]) + (parts[2] + parts[3])
                s23 = (parts[4] + parts[5]) + (parts[6] + parts[7])
                return acc + (s01 + s23)

            acc = plsc.parallel_loop(
                0, nv, 8, carry=jnp.zeros((_L,), jnp.float32))(sb)
            t_s = jnp.max(t_vec)
            res = t_s + jnp.sum(acc) * jnp.float32(1.0 / _K)
            if negate:
                res = -res
            return jnp.where(lanes == lane_off + j, res, res_vec)

        return lax.fori_loop(0, rpw, task, res_vec0)

    res_vec = run(pos_ref, False, 0, jnp.zeros((_L,), jnp.float32))
    res_vec = run(neg_ref, True, rpw, res_vec)
    res_v[...] = res_vec
    pltpu.sync_copy(res_v, out_ref.at[wid])


def _combine(packed, rows):
    rpw = rows // _NW

    def body(x_ref, o_ref):
        x = x_ref[...]
        hp = x[:, 0:rpw]
        hn = x[:, rpw:2 * rpw]
        loss = jnp.maximum(hp - hn + jnp.float32(_MARGIN), 0.0)
        o_ref[...] = jnp.reshape(jnp.sum(loss) * jnp.float32(1.0 / rows), (1, 1))

    return pl.pallas_call(
        body, out_shape=jax.ShapeDtypeStruct((1, 1), jnp.float32))(packed)[0, 0]


@jax.jit
def kernel(positive_distances, negative_distances):
    rows, n = positive_distances.shape
    mesh = plsc.VectorSubcoreMesh(core_axis_name="c", subcore_axis_name="s")
    sc_fn = pl.kernel(
        _sc_topk_body,
        mesh=mesh,
        compiler_params=pltpu.CompilerParams(needs_layout_passes=False),
        out_type=jax.ShapeDtypeStruct((_NW, _L), jnp.float32),
        scratch_types=[
            pltpu.VMEM((2 * n,), jnp.float32),
            pltpu.VMEM((n,), jnp.uint32),
            pltpu.VMEM((_L * 256,), jnp.int32),
            pltpu.SMEM((16,), jnp.int32),
            pltpu.SMEM((16,), jnp.int32),
            pltpu.VMEM((_L,), jnp.float32),
            pltpu.SemaphoreType.DMA((2,)),
        ],
    )
    packed = sc_fn(positive_distances, negative_distances)
    return _combine(packed, rows)


# speedup vs baseline: 1.7036x; 1.1844x over previous
import jax
import jax.numpy as jnp
from jax import lax
from jax.experimental import pallas as pl
from jax.experimental.pallas import tpu as pltpu
from jax.experimental.pallas import tpu_sc as plsc

_MARGIN = 0.2
_K = 64
_L = 16
_NSUB = 16
_NCORE = 2
_NW = _NCORE * _NSUB
_NBUCKETS = 4096


def _sc_topk_body(pos_ref, neg_ref, out_ref, row_v, kb_v, hist_v, tot_v,
                  gtot_v, res_v, sem):
    rows, n = pos_ref.shape
    nv = n // _L
    rpw = rows // _NW
    c = lax.axis_index("c")
    s = lax.axis_index("s")
    wid = c * _NSUB + s
    base = wid * rpw
    lanes = jnp.arange(_L, dtype=jnp.int32)
    ones = jnp.ones((_L,), jnp.int32)
    zeros_i = jnp.zeros((_L,), jnp.int32)

    @plsc.parallel_loop(0, _NBUCKETS // _L, unroll=8)
    def _(i):
        hist_v[pl.ds(i * _L, _L)] = zeros_i

    def scan_top16(smem_ref, r_in, s0):
        def step(i, cgs):
            S, sel, Ssel, found = cgs
            g = 15 - i
            Sn = S + smem_ref[g]
            fn = jnp.logical_and(jnp.logical_not(found), Sn >= r_in)
            sel = jnp.where(fn, g, sel)
            Ssel = jnp.where(fn, S, Ssel)
            return (Sn, sel, Ssel, jnp.logical_or(found, fn))

        _, sel, Ssel, _ = lax.fori_loop(
            0, 16, step, (s0, jnp.int32(0), jnp.int32(0), jnp.bool_(False)),
            unroll=4)
        return sel, Ssel

    def scan_vec16(vg, r_in, s0):
        S = s0
        sel = jnp.int32(0)
        sub = jnp.int32(0)
        totb = jnp.int32(0)
        found = jnp.bool_(False)
        for i in range(_L):
            li = _L - 1 - i
            cnt = vg[li]
            Sn = S + cnt
            fn = jnp.logical_and(jnp.logical_not(found), Sn >= r_in)
            sel = jnp.where(fn, li, sel)
            sub = jnp.where(fn, S, sub)
            totb = jnp.where(fn, cnt, totb)
            found = jnp.logical_or(found, fn)
            S = Sn
        return sel, sub, totb

    def run(src_ref, negate, lane_off, res_vec0):
        pltpu.async_copy(src_ref.at[base], row_v.at[pl.ds(0, n)], sem.at[0])

        def task(j, res_vec):
            pj = lax.rem(j, 2)
            npj = 1 - pj

            @pl.when(j < rpw - 1)
            def _():
                pltpu.async_copy(src_ref.at[base + j + 1],
                                 row_v.at[pl.ds(npj * n, n)], sem.at[npj])

            pltpu.make_async_copy(src_ref.at[base + j],
                                  row_v.at[pl.ds(pj * n, n)], sem.at[pj]).wait()
            off = pj * n

            def load_v(i):
                v = row_v[pl.ds(off + i * _L, _L)]
                return -v if negate else v

            @plsc.parallel_loop(0, nv, unroll=8)
            def _(i):
                u = lax.bitcast_convert_type(load_v(i), jnp.uint32)
                m = jnp.uint32(0x80000000) | (jnp.uint32(0) - (u >> jnp.uint32(31)))
                kb = u ^ m
                kb_v[pl.ds(i * _L, _L)] = kb
                bucket = (kb >> jnp.uint32(20)).astype(jnp.int32)
                plsc.addupdate_scatter(hist_v, [bucket], ones)

            carry = (jnp.int32(_K), jnp.uint32(0), jnp.bool_(False))
            for p in range(3):
                r_in, prefix_in, done_in = carry

                if p == 1:
                    @pl.when(jnp.logical_not(done_in))
                    def _(_prefix=prefix_in):
                        @plsc.parallel_loop(0, nv, unroll=8)
                        def _(i):
                            kb = kb_v[pl.ds(i * _L, _L)]
                            bucket = ((kb >> jnp.uint32(8))
                                      & jnp.uint32(0xFFF)).astype(jnp.int32)
                            active = (kb >> jnp.uint32(20)) == _prefix
                            plsc.addupdate_scatter(hist_v, [bucket], ones,
                                                   mask=active)
                elif p == 2:
                    @pl.when(jnp.logical_not(done_in))
                    def _(_prefix=prefix_in):
                        @plsc.parallel_loop(0, nv, unroll=8)
                        def _(i):
                            kb = kb_v[pl.ds(i * _L, _L)]
                            bucket = (kb & jnp.uint32(0xFF)).astype(jnp.int32)
                            active = (kb >> jnp.uint32(8)) == _prefix
                            plsc.addupdate_scatter(hist_v, [bucket], ones,
                                                   mask=active)

                nb = 256 if p == 2 else _NBUCKETS

                if nb == _NBUCKETS:
                    def redq(q):
                        hs = [hist_v[pl.ds(q * 256 + k * _L, _L)]
                              for k in range(_L)]
                        t01 = (hs[0] + hs[1]) + (hs[2] + hs[3])
                        t23 = (hs[4] + hs[5]) + (hs[6] + hs[7])
                        t45 = (hs[8] + hs[9]) + (hs[10] + hs[11])
                        t67 = (hs[12] + hs[13]) + (hs[14] + hs[15])
                        gtot_v[q] = jnp.sum((t01 + t23) + (t45 + t67))

                    if p == 0:
                        plsc.parallel_loop(0, 16)(redq)
                    else:
                        @pl.when(jnp.logical_not(done_in))
                        def _():
                            plsc.parallel_loop(0, 16)(redq)

                    qsel, s_q = scan_top16(gtot_v, r_in, jnp.int32(0))

                    @plsc.parallel_loop(0, 16)
                    def _(k):
                        tot_v[k] = jnp.sum(hist_v[pl.ds(qsel * 256 + k * _L, _L)])

                    gsel, s_g = scan_top16(tot_v, r_in, s_q)
                    vg = hist_v[pl.ds(qsel * 256 + gsel * _L, _L)]
                    lsel, sub, totb = scan_vec16(vg, r_in, s_g)
                    bsel = qsel * 256 + gsel * _L + lsel
                else:
                    @pl.when(jnp.logical_not(done_in))
                    def _():
                        @plsc.parallel_loop(0, 16)
                        def _(k):
                            gtot_v[k] = jnp.sum(hist_v[pl.ds(k * _L, _L)])

                    gsel, s_g = scan_top16(gtot_v, r_in, jnp.int32(0))
                    vg = hist_v[pl.ds(gsel * _L, _L)]
                    lsel, sub, totb = scan_vec16(vg, r_in, s_g)
                    bsel = gsel * _L + lsel

                @plsc.parallel_loop(0, nb // _L, unroll=8)
                def _(i):
                    hist_v[pl.ds(i * _L, _L)] = zeros_i

                bsel = jnp.where(done_in, 0, bsel)
                sub = jnp.where(done_in, 0, sub)
                r_out = r_in - sub
                width = jnp.uint32(8 if p == 2 else 12)
                carry = (r_out,
                         (prefix_in << width) | bsel.astype(jnp.uint32),
                         jnp.logical_or(done_in, totb == r_out))

            _, kb_t, _ = carry
            kb_vec = jnp.full((_L,), kb_t, dtype=jnp.uint32)
            was_pos = (kb_vec >> jnp.uint32(31)) == jnp.uint32(1)
            bits = jnp.where(was_pos, kb_vec ^ jnp.uint32(0x80000000), ~kb_vec)
            t_vec = lax.bitcast_convert_type(bits, jnp.float32)

            def sb(i, acc):
                parts = [jnp.maximum(load_v(i + k) - t_vec, jnp.float32(0.0))
                         for k in range(8)]
                s01 = (parts[0] + parts[1]) + (parts[2] + parts[3])
                s23 = (parts[4] + parts[5]) + (parts[6] + parts[7])
                return acc + (s01 + s23)

            acc = plsc.parallel_loop(
                0, nv, 8, carry=jnp.zeros((_L,), jnp.float32))(sb)
            t_s = jnp.max(t_vec)
            res = t_s + jnp.sum(acc) * jnp.float32(1.0 / _K)
            if negate:
                res = -res
            return jnp.where(lanes == lane_off + j, res, res_vec)

        return lax.fori_loop(0, rpw, task, res_vec0)

    res_vec = run(pos_ref, False, 0, jnp.zeros((_L,), jnp.float32))
    res_vec = run(neg_ref, True, rpw, res_vec)
    res_v[...] = res_vec
    pltpu.sync_copy(res_v, out_ref.at[wid])


def _combine(packed, rows):
    rpw = rows // _NW

    def body(x_ref, o_ref):
        x = x_ref[...]
        hp = x[:, 0:rpw]
        hn = x[:, rpw:2 * rpw]
        loss = jnp.maximum(hp - hn + jnp.float32(_MARGIN), 0.0)
        o_ref[...] = jnp.reshape(jnp.sum(loss) * jnp.float32(1.0 / rows), (1, 1))

    return pl.pallas_call(
        body, out_shape=jax.ShapeDtypeStruct((1, 1), jnp.float32))(packed)[0, 0]


@jax.jit
def kernel(positive_distances, negative_distances):
    rows, n = positive_distances.shape
    mesh = plsc.VectorSubcoreMesh(core_axis_name="c", subcore_axis_name="s")
    sc_fn = pl.kernel(
        _sc_topk_body,
        mesh=mesh,
        compiler_params=pltpu.CompilerParams(needs_layout_passes=False),
        out_type=jax.ShapeDtypeStruct((_NW, _L), jnp.float32),
        scratch_types=[
            pltpu.VMEM((2 * n,), jnp.float32),
            pltpu.VMEM((n,), jnp.uint32),
            pltpu.VMEM((_NBUCKETS,), jnp.int32),
            pltpu.SMEM((16,), jnp.int32),
            pltpu.SMEM((16,), jnp.int32),
            pltpu.VMEM((_L,), jnp.float32),
            pltpu.SemaphoreType.DMA((2,)),
        ],
    )
    packed = sc_fn(positive_distances, negative_distances)
    return _combine(packed, rows)


# speedup vs baseline: 1.7163x; 1.0075x over previous
import jax
import jax.numpy as jnp
from jax import lax
from jax.experimental import pallas as pl
from jax.experimental.pallas import tpu as pltpu
from jax.experimental.pallas import tpu_sc as plsc

_MARGIN = 0.2
_K = 64
_L = 16
_NSUB = 16
_NCORE = 2
_NW = _NCORE * _NSUB
_NBUCKETS = 4096


def _sc_topk_body(pos_ref, neg_ref, out_ref, row_v, kb_v, hist_v, tot_v,
                  gtot_v, res_v, sem):
    rows, n = pos_ref.shape
    nv = n // _L
    rpw = rows // _NW
    c = lax.axis_index("c")
    s = lax.axis_index("s")
    wid = c * _NSUB + s
    base = wid * rpw
    lanes = jnp.arange(_L, dtype=jnp.int32)
    ones = jnp.ones((_L,), jnp.int32)
    zeros_i = jnp.zeros((_L,), jnp.int32)

    @plsc.parallel_loop(0, _NBUCKETS // _L, unroll=8)
    def _(i):
        hist_v[pl.ds(i * _L, _L)] = zeros_i

    def scan_top16(smem_ref, r_in, s0):
        def step(i, cgs):
            S, sel, Ssel, found = cgs
            g = 15 - i
            Sn = S + smem_ref[g]
            fn = jnp.logical_and(jnp.logical_not(found), Sn >= r_in)
            sel = jnp.where(fn, g, sel)
            Ssel = jnp.where(fn, S, Ssel)
            return (Sn, sel, Ssel, jnp.logical_or(found, fn))

        _, sel, Ssel, _ = lax.fori_loop(
            0, 16, step, (s0, jnp.int32(0), jnp.int32(0), jnp.bool_(False)),
            unroll=4)
        return sel, Ssel

    def scan_vec16(vg, r_in, s0):
        S = s0
        sel = jnp.int32(0)
        sub = jnp.int32(0)
        totb = jnp.int32(0)
        found = jnp.bool_(False)
        for i in range(_L):
            li = _L - 1 - i
            cnt = vg[li]
            Sn = S + cnt
            fn = jnp.logical_and(jnp.logical_not(found), Sn >= r_in)
            sel = jnp.where(fn, li, sel)
            sub = jnp.where(fn, S, sub)
            totb = jnp.where(fn, cnt, totb)
            found = jnp.logical_or(found, fn)
            S = Sn
        return sel, sub, totb

    def run(src_ref, negate, lane_off, res_vec0):
        pltpu.async_copy(src_ref.at[base], row_v.at[pl.ds(0, n)], sem.at[0])

        def task(j, res_vec):
            pj = lax.rem(j, 2)
            npj = 1 - pj

            @pl.when(j < rpw - 1)
            def _():
                pltpu.async_copy(src_ref.at[base + j + 1],
                                 row_v.at[pl.ds(npj * n, n)], sem.at[npj])

            pltpu.make_async_copy(src_ref.at[base + j],
                                  row_v.at[pl.ds(pj * n, n)], sem.at[pj]).wait()
            off = pj * n

            def load_v(i):
                v = row_v[pl.ds(off + i * _L, _L)]
                return -v if negate else v

            @plsc.parallel_loop(0, nv, unroll=16)
            def _(i):
                u = lax.bitcast_convert_type(load_v(i), jnp.uint32)
                m = jnp.uint32(0x80000000) | (jnp.uint32(0) - (u >> jnp.uint32(31)))
                kb = u ^ m
                kb_v[pl.ds(i * _L, _L)] = kb
                bucket = (kb >> jnp.uint32(20)).astype(jnp.int32)
                plsc.addupdate_scatter(hist_v, [bucket], ones)

            carry = (jnp.int32(_K), jnp.uint32(0), jnp.bool_(False))
            for p in range(3):
                r_in, prefix_in, done_in = carry

                if p == 1:
                    @pl.when(jnp.logical_not(done_in))
                    def _(_prefix=prefix_in):
                        @plsc.parallel_loop(0, nv, unroll=16)
                        def _(i):
                            kb = kb_v[pl.ds(i * _L, _L)]
                            bucket = ((kb >> jnp.uint32(8))
                                      & jnp.uint32(0xFFF)).astype(jnp.int32)
                            active = (kb >> jnp.uint32(20)) == _prefix
                            plsc.addupdate_scatter(hist_v, [bucket], ones,
                                                   mask=active)
                elif p == 2:
                    @pl.when(jnp.logical_not(done_in))
                    def _(_prefix=prefix_in):
                        @plsc.parallel_loop(0, nv, unroll=16)
                        def _(i):
                            kb = kb_v[pl.ds(i * _L, _L)]
                            bucket = (kb & jnp.uint32(0xFF)).astype(jnp.int32)
                            active = (kb >> jnp.uint32(8)) == _prefix
                            plsc.addupdate_scatter(hist_v, [bucket], ones,
                                                   mask=active)

                nb = 256 if p == 2 else _NBUCKETS

                if nb == _NBUCKETS:
                    def redq(q):
                        hs = [hist_v[pl.ds(q * 256 + k * _L, _L)]
                              for k in range(_L)]
                        t01 = (hs[0] + hs[1]) + (hs[2] + hs[3])
                        t23 = (hs[4] + hs[5]) + (hs[6] + hs[7])
                        t45 = (hs[8] + hs[9]) + (hs[10] + hs[11])
                        t67 = (hs[12] + hs[13]) + (hs[14] + hs[15])
                        gtot_v[q] = jnp.sum((t01 + t23) + (t45 + t67))

                    if p == 0:
                        plsc.parallel_loop(0, 16)(redq)
                    else:
                        @pl.when(jnp.logical_not(done_in))
                        def _():
                            plsc.parallel_loop(0, 16)(redq)

                    qsel, s_q = scan_top16(gtot_v, r_in, jnp.int32(0))

                    @plsc.parallel_loop(0, 16)
                    def _(k):
                        tot_v[k] = jnp.sum(hist_v[pl.ds(qsel * 256 + k * _L, _L)])

                    gsel, s_g = scan_top16(tot_v, r_in, s_q)
                    vg = hist_v[pl.ds(qsel * 256 + gsel * _L, _L)]
                    lsel, sub, totb = scan_vec16(vg, r_in, s_g)
                    bsel = qsel * 256 + gsel * _L + lsel
                else:
                    @pl.when(jnp.logical_not(done_in))
                    def _():
                        @plsc.parallel_loop(0, 16)
                        def _(k):
                            gtot_v[k] = jnp.sum(hist_v[pl.ds(k * _L, _L)])

                    gsel, s_g = scan_top16(gtot_v, r_in, jnp.int32(0))
                    vg = hist_v[pl.ds(gsel * _L, _L)]
                    lsel, sub, totb = scan_vec16(vg, r_in, s_g)
                    bsel = gsel * _L + lsel

                @plsc.parallel_loop(0, nb // _L, unroll=8)
                def _(i):
                    hist_v[pl.ds(i * _L, _L)] = zeros_i

                bsel = jnp.where(done_in, 0, bsel)
                sub = jnp.where(done_in, 0, sub)
                r_out = r_in - sub
                width = jnp.uint32(8 if p == 2 else 12)
                carry = (r_out,
                         (prefix_in << width) | bsel.astype(jnp.uint32),
                         jnp.logical_or(done_in, totb == r_out))

            _, kb_t, _ = carry
            kb_vec = jnp.full((_L,), kb_t, dtype=jnp.uint32)
            was_pos = (kb_vec >> jnp.uint32(31)) == jnp.uint32(1)
            bits = jnp.where(was_pos, kb_vec ^ jnp.uint32(0x80000000), ~kb_vec)
            t_vec = lax.bitcast_convert_type(bits, jnp.float32)

            def sb(i, acc):
                parts = [jnp.maximum(load_v(i + k) - t_vec, jnp.float32(0.0))
                         for k in range(8)]
                s01 = (parts[0] + parts[1]) + (parts[2] + parts[3])
                s23 = (parts[4] + parts[5]) + (parts[6] + parts[7])
                return acc + (s01 + s23)

            acc = plsc.parallel_loop(
                0, nv, 8, carry=jnp.zeros((_L,), jnp.float32))(sb)
            t_s = jnp.max(t_vec)
            res = t_s + jnp.sum(acc) * jnp.float32(1.0 / _K)
            if negate:
                res = -res
            return jnp.where(lanes == lane_off + j, res, res_vec)

        return lax.fori_loop(0, rpw, task, res_vec0)

    res_vec = run(pos_ref, False, 0, jnp.zeros((_L,), jnp.float32))
    res_vec = run(neg_ref, True, rpw, res_vec)
    res_v[...] = res_vec
    pltpu.sync_copy(res_v, out_ref.at[wid])


def _combine(packed, rows):
    rpw = rows // _NW

    def body(x_ref, o_ref):
        x = x_ref[...]
        hp = x[:, 0:rpw]
        hn = x[:, rpw:2 * rpw]
        loss = jnp.maximum(hp - hn + jnp.float32(_MARGIN), 0.0)
        o_ref[...] = jnp.reshape(jnp.sum(loss) * jnp.float32(1.0 / rows), (1, 1))

    return pl.pallas_call(
        body, out_shape=jax.ShapeDtypeStruct((1, 1), jnp.float32))(packed)[0, 0]


@jax.jit
def kernel(positive_distances, negative_distances):
    rows, n = positive_distances.shape
    mesh = plsc.VectorSubcoreMesh(core_axis_name="c", subcore_axis_name="s")
    sc_fn = pl.kernel(
        _sc_topk_body,
        mesh=mesh,
        compiler_params=pltpu.CompilerParams(needs_layout_passes=False),
        out_type=jax.ShapeDtypeStruct((_NW, _L), jnp.float32),
        scratch_types=[
            pltpu.VMEM((2 * n,), jnp.float32),
            pltpu.VMEM((n,), jnp.uint32),
            pltpu.VMEM((_NBUCKETS,), jnp.int32),
            pltpu.SMEM((16,), jnp.int32),
            pltpu.SMEM((16,), jnp.int32),
            pltpu.VMEM((_L,), jnp.float32),
            pltpu.SemaphoreType.DMA((2,)),
        ],
    )
    packed = sc_fn(positive_distances, negative_distances)
    return _combine(packed, rows)
